# Initial kernel scaffold; baseline (speedup 1.0000x reference)
import jax
import jax.numpy as jnp
from jax.experimental import pallas as pl


def kernel(s_coor, s_mask, d_coor):
    raise NotImplementedError("write your pallas kernel here")


# speedup vs baseline: 4.8565x; 4.8565x over previous
import functools

import jax
import jax.numpy as jnp
from jax import lax
from jax.experimental import pallas as pl

K = 16
NVAL = 8


def _body(nk, block_q, d_ref, st_ref, w_ref, o_ref):
    d = d_ref[...]
    st = st_ref[...]
    m = st.shape[1]
    d_sq = jnp.sum(d * d, axis=1, keepdims=True)
    s_sq = jnp.sum(st * st, axis=0, keepdims=True)
    dot = lax.dot_general(d, st, (((1,), (0,)), ((), ())),
                          preferred_element_type=jnp.float32)
    dist = d_sq + s_sq - 2.0 * dot

    iota = lax.broadcasted_iota(jnp.int32, (block_q, m), 1)
    sel = jnp.zeros((block_q, m), jnp.float32)
    for _ in range(nk):
        mn = jnp.min(dist, axis=1, keepdims=True)
        eq = dist == mn
        am = jnp.min(jnp.where(eq, iota, m), axis=1, keepdims=True)
        oh = iota == am
        sel = jnp.where(oh, 1.0, sel)
        dist = jnp.where(oh, jnp.inf, dist)

    counts = lax.dot_general(sel.astype(jnp.bfloat16), w_ref[...],
                             (((1,), (0,)), ((), ())),
                             preferred_element_type=jnp.float32)
    ci = counts.astype(jnp.int32)
    v = lax.broadcasted_iota(jnp.int32, ci.shape, 1) % NVAL
    score = ci * NVAL + (NVAL - 1 - v)
    best = jnp.max(score.reshape(block_q, -1, NVAL), axis=2)
    o_ref[...] = (NVAL - 1 - best % NVAL).astype(jnp.float32)


def kernel(s_coor, s_mask, d_coor):
    mkeys, dim = s_coor.shape
    n = d_coor.shape[0]
    dmask = s_mask.shape[1]
    block_q = 256 if n % 256 == 0 else n
    grid = n // block_q

    s_t = s_coor.T
    w = (s_mask[:, :, None] == jnp.arange(NVAL, dtype=s_mask.dtype)
         ).reshape(mkeys, dmask * NVAL).astype(jnp.bfloat16)

    out = pl.pallas_call(
        functools.partial(_body, K, block_q),
        grid=(grid,),
        in_specs=[
            pl.BlockSpec((block_q, dim), lambda i: (i, 0)),
            pl.BlockSpec((dim, mkeys), lambda i: (0, 0)),
            pl.BlockSpec((mkeys, dmask * NVAL), lambda i: (0, 0)),
        ],
        out_specs=pl.BlockSpec((block_q, dmask), lambda i: (i, 0)),
        out_shape=jax.ShapeDtypeStruct((n, dmask), jnp.float32),
    )(d_coor, s_t, w)
    return out
